# baseline (speedup 1.0000x reference)
import functools

import jax
import jax.numpy as jnp
from jax import lax
from jax.experimental import pallas as pl
from jax.experimental.pallas import tpu as pltpu
from jax.experimental.pallas import tpu_sc as plsc

_NROWS = 16384
_NCOLS = 200
_L = 16


def _sc_energy_shift(species_flat, energies, se_pad):
    info = plsc.get_sparse_core_info()
    nw = info.num_cores * info.num_subcores
    rows_w = _NROWS // nw
    groups = rows_w // _L
    jsteps = _NCOLS // 4

    mesh = plsc.VectorSubcoreMesh(core_axis_name="c", subcore_axis_name="s")

    @functools.partial(
        pl.kernel,
        mesh=mesh,
        out_type=jax.ShapeDtypeStruct((_NROWS,), jnp.float32),
        compiler_params=pltpu.CompilerParams(needs_layout_passes=False),
        scratch_types=[
            pltpu.VMEM((rows_w * _NCOLS,), jnp.int32),
            pltpu.VMEM((4096,), jnp.float32),
            pltpu.VMEM((_L,), jnp.float32),
            pltpu.VMEM((rows_w,), jnp.float32),
            pltpu.VMEM((rows_w,), jnp.float32),
        ],
    )
    def k(species_hbm, energies_hbm, se_hbm, out_hbm, sp_v, tb_v, se_v, en_v, out_v):
        wid = lax.axis_index("s") * info.num_cores + lax.axis_index("c")
        rbase = wid * rows_w
        pltpu.sync_copy(se_hbm, se_v)
        pltpu.sync_copy(energies_hbm.at[pl.ds(rbase, rows_w)], en_v)
        pltpu.sync_copy(species_hbm.at[pl.ds(rbase * _NCOLS, rows_w * _NCOLS)], sp_v)

        lanes = lax.iota(jnp.int32, _L)

        def build(i, carry):
            e = i * _L + lanes
            a = lax.shift_right_logical(e, 9)
            b = lax.shift_right_logical(e, 6) & 7
            c = lax.shift_right_logical(e, 3) & 7
            d = e & 7
            t = (plsc.load_gather(se_v, [a]) + plsc.load_gather(se_v, [b])
                 + plsc.load_gather(se_v, [c]) + plsc.load_gather(se_v, [d]))
            tb_v[pl.ds(i * _L, _L)] = t
            return carry

        lax.fori_loop(0, 4096 // _L, build, 0)

        def group(g, carry):
            rowoff = (g * _L + lanes) * _NCOLS

            def step(j, acc):
                i0 = rowoff + j * 4
                s0 = plsc.load_gather(sp_v, [i0])
                s1 = plsc.load_gather(sp_v, [i0 + 1])
                s2 = plsc.load_gather(sp_v, [i0 + 2])
                s3 = plsc.load_gather(sp_v, [i0 + 3])
                idx = ((s0 * 8 + s1) * 8 + s2) * 8 + s3
                return acc + plsc.load_gather(tb_v, [idx])

            acc = lax.fori_loop(0, jsteps, step, jnp.zeros((_L,), jnp.float32))
            out_v[pl.ds(g * _L, _L)] = acc + en_v[pl.ds(g * _L, _L)]
            return carry

        lax.fori_loop(0, groups, group, 0)
        pltpu.sync_copy(out_v, out_hbm.at[pl.ds(rbase, rows_w)])

    return k(species_flat, energies, se_pad)


def kernel(species, energies, self_energies):
    sp_flat = species.reshape(-1).astype(jnp.int32)
    se_pad = jnp.concatenate(
        [self_energies.astype(jnp.float32),
         jnp.zeros((_L - self_energies.shape[0],), jnp.float32)]
    )
    new_energies = _sc_energy_shift(sp_flat, energies.astype(jnp.float32), se_pad)
    return (species, new_energies)
